# baseline (speedup 1.0000x reference)
import jax
import jax.numpy as jnp
from jax.experimental import pallas as pl
from jax.experimental.pallas import tpu as pltpu

_B, _M, _N = 4, 4096, 2048
_OUT = (_B, _M * _N)

_CHUNKS = 4
_ROWS = _M // _CHUNKS


def _copy_body(x_ref, o_ref, sems):
    for b in range(_B):
        for c in range(_CHUNKS):
            pltpu.make_async_copy(
                x_ref.at[b, pl.ds(c * _ROWS, _ROWS)],
                o_ref.at[b, pl.ds(c * _ROWS, _ROWS)],
                sems.at[b, c],
            ).start()
    for b in range(_B):
        for c in range(_CHUNKS):
            pltpu.make_async_copy(
                x_ref.at[b, pl.ds(c * _ROWS, _ROWS)],
                o_ref.at[b, pl.ds(c * _ROWS, _ROWS)],
                sems.at[b, c],
            ).wait()


def kernel(tensor):
    out = pl.pallas_call(
        _copy_body,
        out_shape=jax.ShapeDtypeStruct((_B, _M, _N), tensor.dtype),
        in_specs=[pl.BlockSpec(memory_space=pl.ANY)],
        out_specs=pl.BlockSpec(memory_space=pl.ANY),
        scratch_shapes=[pltpu.SemaphoreType.DMA((_B, _CHUNKS))],
    )(tensor)
    return jnp.reshape(out, _OUT)


# speedup vs baseline: 21.6050x; 21.6050x over previous
import jax
import jax.numpy as jnp
from jax.experimental import pallas as pl
from jax.experimental.pallas import tpu as pltpu

_B, _M, _N = 4, 4096, 2048
_OUT = (_B, _M * _N)

_BLK_M = 512


def _copy_body(x_ref, o_ref):
    o_ref[...] = x_ref[...]


def kernel(tensor):
    out = pl.pallas_call(
        _copy_body,
        grid=(_B, _M // _BLK_M),
        in_specs=[pl.BlockSpec((1, _BLK_M, _N), lambda b, m: (b, m, 0))],
        out_specs=pl.BlockSpec((1, _BLK_M, _N), lambda b, m: (b, m, 0)),
        out_shape=jax.ShapeDtypeStruct((_B, _M, _N), tensor.dtype),
        compiler_params=pltpu.CompilerParams(
            dimension_semantics=("arbitrary", "arbitrary"),
        ),
    )(tensor)
    return jnp.reshape(out, _OUT)


# speedup vs baseline: 21.7042x; 1.0046x over previous
import jax
import jax.numpy as jnp
from jax.experimental import pallas as pl
from jax.experimental.pallas import tpu as pltpu

_B, _M, _N = 4, 4096, 2048
_OUT = (_B, _M * _N)

_BLK_M = 256
_CPB = _M // _BLK_M
_CHUNKS = _B * _CPB
_NBUF = 8
_DEPTH = 4


def _copy_body(x_ref, o_ref, buf, in_sems, out_sems):
    def in_copy(c):
        bt, r = divmod(c, _CPB)
        i = c % _NBUF
        return pltpu.make_async_copy(
            x_ref.at[bt, pl.ds(r * _BLK_M, _BLK_M)], buf.at[i], in_sems.at[i])

    def out_copy(c):
        bt, r = divmod(c, _CPB)
        i = c % _NBUF
        return pltpu.make_async_copy(
            buf.at[i], o_ref.at[bt, pl.ds(r * _BLK_M, _BLK_M)], out_sems.at[i])

    for c in range(_DEPTH):
        in_copy(c).start()
    for c in range(_CHUNKS):
        pf = c + _DEPTH
        if pf < _CHUNKS:
            if pf >= _NBUF:
                out_copy(pf - _NBUF).wait()
            in_copy(pf).start()
        in_copy(c).wait()
        out_copy(c).start()
    for c in range(_CHUNKS - _NBUF, _CHUNKS):
        out_copy(c).wait()


def kernel(tensor):
    out = pl.pallas_call(
        _copy_body,
        out_shape=jax.ShapeDtypeStruct((_B, _M, _N), tensor.dtype),
        in_specs=[pl.BlockSpec(memory_space=pl.ANY)],
        out_specs=pl.BlockSpec(memory_space=pl.ANY),
        scratch_shapes=[
            pltpu.VMEM((_NBUF, _BLK_M, _N), tensor.dtype),
            pltpu.SemaphoreType.DMA((_NBUF,)),
            pltpu.SemaphoreType.DMA((_NBUF,)),
        ],
    )(tensor)
    return jnp.reshape(out, _OUT)
